# baseline (speedup 1.0000x reference)
import functools

import jax
import jax.numpy as jnp
from jax import lax
from jax.experimental import pallas as pl
from jax.experimental.pallas import tpu as pltpu
from jax.experimental.pallas import tpu_sc as plsc

VOCAB = 100000
EMBED_DIM = 64
BATCH = 4096
HIST = 200

NC = 2
NS = 16
NW = NC * NS

TOTAL = BATCH * HIST
PER_W = TOTAL // NW
CHUNK = 320
NCHUNK = PER_W // CHUNK
NBUF = 4
K = 2

R_STAGE = 26112
RPW = R_STAGE // NS
SENT = jnp.int32(0x7FFFFFFF)

assert PER_W % CHUNK == 0 and NCHUNK % NBUF == 0 and 0 < K < NBUF
assert CHUNK % 16 == 0 and R_STAGE % (8 * NS) == 0

_mesh = plsc.VectorSubcoreMesh(
    core_axis_name="c", subcore_axis_name="s", num_cores=NC, num_subcores=NS
)


@functools.partial(
    pl.kernel,
    out_type=jax.ShapeDtypeStruct((TOTAL, EMBED_DIM), jnp.float32),
    mesh=_mesh,
    scratch_types=[
        pltpu.VMEM((PER_W,), jnp.int32),
        [pltpu.VMEM((CHUNK, EMBED_DIM), jnp.float32)] * NBUF,
        [pltpu.VMEM((CHUNK,), jnp.int32)] * NBUF,
        [pltpu.VMEM((CHUNK,), jnp.int32)] * NBUF,
        [pltpu.SemaphoreType.DMA] * NBUF,
        [pltpu.SemaphoreType.DMA] * NBUF,
        [pltpu.SemaphoreType.DMA] * NBUF,
    ],
    compiler_params=pltpu.CompilerParams(use_tc_tiling_on_sc=False),
)
def _gather_kernel(idx_hbm, table_hbm, out_hbm, idx_v, rows,
                   idx_lo, idx_hi, gsem_lo, gsem_hi, ssem):
    sid = lax.axis_index("s")
    wid = sid * NC + lax.axis_index("c")
    wbase = wid * PER_W

    def prep(j, b):
        base = j * CHUNK

        def vstep(i, carry):
            v = idx_v[pl.ds(base + i * 16, 16)]
            m = v < R_STAGE
            idx_lo[b][pl.ds(i * 16, 16)] = jnp.where(m, v, SENT)
            idx_hi[b][pl.ds(i * 16, 16)] = jnp.where(m, SENT, v)
            return carry

        lax.fori_loop(0, CHUNK // 16, vstep, 0)

    def start_gather(g, b):
        pltpu.async_copy(
            table_hbm.at[plsc.Indices(idx_lo[b], ignored_value=0x7FFFFFFF)],
            rows[b], gsem_lo[b])
        pltpu.async_copy(
            table_hbm.at[plsc.Indices(idx_hi[b], ignored_value=0x7FFFFFFF)],
            rows[b], gsem_hi[b])

    def wait_gather(b):
        pltpu.make_async_copy(
            table_hbm.at[plsc.Indices(idx_lo[b], ignored_value=0x7FFFFFFF)],
            rows[b], gsem_lo[b]).wait()
        pltpu.make_async_copy(
            table_hbm.at[plsc.Indices(idx_hi[b], ignored_value=0x7FFFFFFF)],
            rows[b], gsem_hi[b]).wait()

    def start_scatter(g, b):
        pltpu.async_copy(
            rows[b], out_hbm.at[pl.ds(wbase + g * CHUNK, CHUNK)], ssem[b])

    def wait_scatter(b):
        pltpu.make_async_copy(
            rows[b], out_hbm.at[pl.ds(wbase, CHUNK)], ssem[b]).wait()

    pltpu.sync_copy(idx_hbm.at[pl.ds(wbase, PER_W)], idx_v)

    for j in range(K):
        prep(j, j % NBUF)
        start_gather(j, j % NBUF)

    for g in range(NBUF - K):
        b = g % NBUF
        wait_gather(b)
        start_scatter(g, b)
        prep(g + K, (g + K) % NBUF)
        start_gather(g + K, (g + K) % NBUF)

    def step(h, carry):
        for t in range(NBUF):
            g = (NBUF - K) + h * NBUF + t
            b = (NBUF - K + t) % NBUF
            bj = (b + K) % NBUF
            wait_gather(b)
            start_scatter(g, b)
            wait_scatter(bj)
            prep(g + K, bj)
            start_gather(g + K, bj)
        return carry

    lax.fori_loop(0, (NCHUNK - NBUF) // NBUF, step, 0)

    for g in range(NCHUNK - K, NCHUNK):
        b = g % NBUF
        wait_gather(b)
        start_scatter(g, b)
    for b in range(NBUF):
        wait_scatter(b)


def kernel(words, W):
    idx = words.reshape(TOTAL).astype(jnp.int32)
    out = _gather_kernel(idx, W)
    return out.reshape(BATCH, HIST, EMBED_DIM)


# speedup vs baseline: 1.0007x; 1.0007x over previous
import functools

import jax
import jax.numpy as jnp
from jax import lax
from jax.experimental import pallas as pl
from jax.experimental.pallas import tpu as pltpu
from jax.experimental.pallas import tpu_sc as plsc

VOCAB = 100000
EMBED_DIM = 64
BATCH = 4096
HIST = 200

NC = 2
NS = 16
NW = NC * NS

TOTAL = BATCH * HIST
PER_W = TOTAL // NW
CHUNK = 160
NCHUNK = PER_W // CHUNK
NBUF = 4
K = 2
K2 = 3

R_STAGE = 15488
RPW = R_STAGE // NS
SENT = jnp.int32(0x7FFFFFFF)

assert PER_W % CHUNK == 0 and CHUNK % 16 == 0
assert 0 < K < K2 <= NBUF
assert R_STAGE % (8 * NS) == 0

_mesh = plsc.VectorSubcoreMesh(
    core_axis_name="c", subcore_axis_name="s", num_cores=NC, num_subcores=NS
)


@functools.partial(
    pl.kernel,
    out_type=jax.ShapeDtypeStruct((TOTAL, EMBED_DIM), jnp.float32),
    mesh=_mesh,
    scratch_types=[
        pltpu.VMEM_SHARED((R_STAGE, EMBED_DIM), jnp.float32),
        [pltpu.VMEM((CHUNK, EMBED_DIM), jnp.float32)] * NBUF,
        [pltpu.VMEM((CHUNK,), jnp.int32)] * NBUF,
        [pltpu.VMEM((CHUNK,), jnp.int32)] * NBUF,
        [pltpu.VMEM((CHUNK,), jnp.int32)] * NBUF,
        [pltpu.SemaphoreType.DMA] * NBUF,
        [pltpu.SemaphoreType.DMA] * NBUF,
        [pltpu.SemaphoreType.DMA] * NBUF,
        [pltpu.SemaphoreType.DMA] * NBUF,
    ],
    compiler_params=pltpu.CompilerParams(use_tc_tiling_on_sc=False),
)
def _gather_kernel(idx_hbm, table_hbm, out_hbm, spm, rows, idxraw,
                   idx_lo, idx_hi, isem, gsem_lo, gsem_hi, ssem):
    sid = lax.axis_index("s")
    wid = sid * NC + lax.axis_index("c")
    wbase = wid * PER_W

    def start_idx(j, b):
        pltpu.async_copy(
            idx_hbm.at[pl.ds(wbase + j * CHUNK, CHUNK)], idxraw[b], isem[b])

    def wait_idx(b):
        pltpu.make_async_copy(
            idx_hbm.at[pl.ds(wbase, CHUNK)], idxraw[b], isem[b]).wait()

    def prep(b):
        def vstep(i, carry):
            v = idxraw[b][pl.ds(i * 16, 16)]
            m = v < R_STAGE
            idx_lo[b][pl.ds(i * 16, 16)] = jnp.where(m, v, SENT)
            idx_hi[b][pl.ds(i * 16, 16)] = jnp.where(m, SENT, v)
            return carry

        lax.fori_loop(0, CHUNK // 16, vstep, 0)

    def start_gather(b):
        pltpu.async_copy(
            spm.at[plsc.Indices(idx_lo[b], ignored_value=0x7FFFFFFF)],
            rows[b], gsem_lo[b])
        pltpu.async_copy(
            table_hbm.at[plsc.Indices(idx_hi[b], ignored_value=0x7FFFFFFF)],
            rows[b], gsem_hi[b])

    def wait_gather(b):
        pltpu.make_async_copy(
            spm.at[plsc.Indices(idx_lo[b], ignored_value=0x7FFFFFFF)],
            rows[b], gsem_lo[b]).wait()
        pltpu.make_async_copy(
            table_hbm.at[plsc.Indices(idx_hi[b], ignored_value=0x7FFFFFFF)],
            rows[b], gsem_hi[b]).wait()

    def start_scatter(g, b):
        pltpu.async_copy(
            rows[b], out_hbm.at[pl.ds(wbase + g * CHUNK, CHUNK)], ssem[b])

    def wait_scatter(b):
        pltpu.make_async_copy(
            rows[b], out_hbm.at[pl.ds(wbase, CHUNK)], ssem[b]).wait()

    pltpu.sync_copy(table_hbm.at[pl.ds(sid * RPW, RPW)],
                    spm.at[pl.ds(sid * RPW, RPW)])
    plsc.subcore_barrier()

    for j in range(K2):
        start_idx(j, j % NBUF)
    for j in range(K):
        wait_idx(j % NBUF)
        prep(j % NBUF)
        start_gather(j % NBUF)

    def pipeline_step(g, b, idx_ok, scat_ok):
        wait_gather(b)
        start_scatter(g, b)
        if idx_ok:
            start_idx(g + K2, (b + K2) % NBUF)
        bj = (b + K) % NBUF
        if scat_ok:
            wait_scatter(bj)
        wait_idx(bj)
        prep(bj)
        start_gather(bj)

    for g in range(NBUF - K):
        pipeline_step(g, g % NBUF, True, False)

    STEADY0 = NBUF - K
    NSTEADY = ((NCHUNK - K2 - STEADY0) // NBUF) * NBUF

    def step(h, carry):
        for t in range(NBUF):
            g = STEADY0 + h * NBUF + t
            pipeline_step(g, (STEADY0 + t) % NBUF, True, True)
        return carry

    lax.fori_loop(0, NSTEADY // NBUF, step, 0)

    for g in range(STEADY0 + NSTEADY, NCHUNK - K):
        pipeline_step(g, g % NBUF, g + K2 < NCHUNK, True)
    for g in range(NCHUNK - K, NCHUNK):
        b = g % NBUF
        wait_gather(b)
        start_scatter(g, b)
    for b in range(NBUF):
        wait_scatter(b)


def kernel(words, W):
    idx = words.reshape(TOTAL).astype(jnp.int32)
    out = _gather_kernel(idx, W)
    return out.reshape(BATCH, HIST, EMBED_DIM)


# speedup vs baseline: 1.0032x; 1.0025x over previous
import functools

import jax
import jax.numpy as jnp
from jax import lax
from jax.experimental import pallas as pl
from jax.experimental.pallas import tpu as pltpu
from jax.experimental.pallas import tpu_sc as plsc

VOCAB = 100000
EMBED_DIM = 64
BATCH = 4096
HIST = 200

NC = 2
NS = 16
NW = NC * NS

TOTAL = BATCH * HIST
PER_W = TOTAL // NW
CHUNK = 128
NCHUNK = PER_W // CHUNK
NBUF = 8
K = 4

assert PER_W % CHUNK == 0 and NCHUNK % NBUF == 0 and 0 < K < NBUF

_mesh = plsc.VectorSubcoreMesh(
    core_axis_name="c", subcore_axis_name="s", num_cores=NC, num_subcores=NS
)


@functools.partial(
    pl.kernel,
    out_type=jax.ShapeDtypeStruct((TOTAL, EMBED_DIM), jnp.float32),
    mesh=_mesh,
    scratch_types=[
        pltpu.VMEM((PER_W,), jnp.int32),
        [pltpu.VMEM((CHUNK, EMBED_DIM), jnp.float32)] * NBUF,
        [pltpu.SemaphoreType.DMA] * NBUF,
        [pltpu.SemaphoreType.DMA] * NBUF,
    ],
    compiler_params=pltpu.CompilerParams(use_tc_tiling_on_sc=False),
)
def _gather_kernel(idx_hbm, table_hbm, out_hbm, idx_v, rows, gsem, ssem):
    wid = lax.axis_index("s") * NC + lax.axis_index("c")
    wbase = wid * PER_W

    def start_gather(g, b):
        pltpu.async_copy(
            table_hbm.at[idx_v.at[pl.ds(g * CHUNK, CHUNK)]], rows[b], gsem[b])

    def wait_gather(b):
        pltpu.make_async_copy(
            table_hbm.at[idx_v.at[pl.ds(0, CHUNK)]], rows[b], gsem[b]).wait()

    def start_scatter(g, b):
        pltpu.async_copy(
            rows[b], out_hbm.at[pl.ds(wbase + g * CHUNK, CHUNK)], ssem[b])

    def wait_scatter(b):
        pltpu.make_async_copy(
            rows[b], out_hbm.at[pl.ds(wbase, CHUNK)], ssem[b]).wait()

    pltpu.sync_copy(idx_hbm.at[pl.ds(wbase, PER_W)], idx_v)

    for j in range(K):
        start_gather(j, j % NBUF)

    for g in range(NBUF - K):
        b = g % NBUF
        wait_gather(b)
        start_scatter(g, b)
        start_gather(g + K, (g + K) % NBUF)

    def step(h, carry):
        for t in range(NBUF):
            g = (NBUF - K) + h * NBUF + t
            b = (NBUF - K + t) % NBUF
            bj = (b + K) % NBUF
            wait_gather(b)
            start_scatter(g, b)
            wait_scatter(bj)
            start_gather(g + K, bj)
        return carry

    lax.fori_loop(0, (NCHUNK - NBUF) // NBUF, step, 0)

    for g in range(NCHUNK - K, NCHUNK):
        b = g % NBUF
        wait_gather(b)
        start_scatter(g, b)
    for b in range(NBUF):
        wait_scatter(b)


def kernel(words, W):
    idx = words.reshape(TOTAL).astype(jnp.int32)
    out = _gather_kernel(idx, W)
    return out.reshape(BATCH, HIST, EMBED_DIM)
